# baseline (speedup 1.0000x reference)
import functools

import jax
import jax.numpy as jnp
from jax import lax
from jax.experimental import pallas as pl
from jax.experimental.pallas import tpu as pltpu
from jax.experimental.pallas import tpu_sc as plsc

N = 10000
E = 320000
D = 128
KG = 3
PDIM = 4
EPS = 1e-15

NC = 2
NS = 16
LANES = 16
DH = D // NC
GCOLS = KG * DH
MW = DH + LANES
CHUNK = 128
NCHUNKS = E // CHUNK
BASE_CPT = NCHUNKS // NS
TAIL = NCHUNKS - BASE_CPT * NS
SUP = 12
NSUP = BASE_CPT // SUP
NPAD = 10240
ROWS_SUB = NPAD // NS
EGROUPS = CHUNK // LANES


def _tc_pre_body(vals_ref, gp_ref, root_ref, dense_ref, bias_ref, pseudo_ref,
                 mu0_ref, s0_ref, mu1_ref, s1_ref,
                 xg_ref, r_ref, w0_ref, w1_ref):
    x = vals_ref[...]
    xgfull = jnp.dot(x, gp_ref[...],
                     preferred_element_type=jnp.float32).astype(jnp.bfloat16)
    xg_ref[0] = xgfull[:, 0:GCOLS]
    xg_ref[1] = xgfull[:, GCOLS:2 * GCOLS]
    r_ref[...] = (jnp.dot(x, root_ref[...] + dense_ref[...],
                          preferred_element_type=jnp.float32) + bias_ref[...])
    for mu_ref, s_ref, w_ref in ((mu0_ref, s0_ref, w0_ref),
                                 (mu1_ref, s1_ref, w1_ref)):
        mu = mu_ref[...]
        sg = s_ref[...]
        for k in range(KG):
            acc = None
            for dd in range(PDIM):
                pd = pseudo_ref[dd]
                mkd = mu[k:k + 1, dd:dd + 1]
                skd = sg[k:k + 1, dd:dd + 1]
                t = (pd - mkd) ** 2 * (-0.5 / (EPS + skd * skd))
                acc = t if acc is None else acc + t
            w_ref[k] = jnp.exp(acc)


_tc_pre = pl.pallas_call(
    _tc_pre_body,
    out_shape=[
        jax.ShapeDtypeStruct((NC, N, GCOLS), jnp.bfloat16),
        jax.ShapeDtypeStruct((N, D), jnp.float32),
        jax.ShapeDtypeStruct((KG, E // 128, 128), jnp.float32),
        jax.ShapeDtypeStruct((KG, E // 128, 128), jnp.float32),
    ],
)


def _combine_bn(agg_ref, r_ref, gamma_ref, beta_ref):
    feat = jnp.concatenate(
        [agg_ref[0:N, 0:DH], agg_ref[0:N, MW:MW + DH]], axis=1)
    cnt = agg_ref[0:N, DH:DH + 1]
    y = feat / jnp.maximum(cnt, 1.0) + r_ref[...]
    m = jnp.mean(y, axis=0, keepdims=True)
    v = jnp.mean((y - m) ** 2, axis=0, keepdims=True)
    return gamma_ref[...] * ((y - m) / jnp.sqrt(v + 1e-5)) + beta_ref[...]


def _tc_mid_body(agg_ref, r_ref, gamma_ref, beta_ref,
                 gp_ref, root_ref, dense_ref, bias_ref, xg_ref, rout_ref):
    y = _combine_bn(agg_ref, r_ref, gamma_ref, beta_ref)
    x1 = jnp.maximum(y, 0.0)
    xgfull = jnp.dot(x1, gp_ref[...],
                     preferred_element_type=jnp.float32).astype(jnp.bfloat16)
    xg_ref[0] = xgfull[:, 0:GCOLS]
    xg_ref[1] = xgfull[:, GCOLS:2 * GCOLS]
    rout_ref[...] = (jnp.dot(x1, root_ref[...] + dense_ref[...],
                             preferred_element_type=jnp.float32) + bias_ref[...])


_tc_mid = pl.pallas_call(
    _tc_mid_body,
    out_shape=[
        jax.ShapeDtypeStruct((NC, N, GCOLS), jnp.bfloat16),
        jax.ShapeDtypeStruct((N, D), jnp.float32),
    ],
)


def _tc_post_body(agg_ref, r_ref, gamma_ref, beta_ref, out_ref):
    out_ref[...] = _combine_bn(agg_ref, r_ref, gamma_ref, beta_ref)


_tc_post = pl.pallas_call(
    _tc_post_body,
    out_shape=jax.ShapeDtypeStruct((N, D), jnp.float32),
)


_SC_MESH = plsc.VectorSubcoreMesh(core_axis_name="c", subcore_axis_name="s")


@functools.partial(
    pl.kernel,
    out_type=jax.ShapeDtypeStruct((NPAD, NC * MW), jnp.float32),
    mesh=_SC_MESH,
    compiler_params=pltpu.CompilerParams(use_tc_tiling_on_sc=False,
                                         needs_layout_passes=False),
    scratch_types=[
        pltpu.VMEM((2, SUP, CHUNK), jnp.int32),
        pltpu.VMEM((2, SUP, CHUNK), jnp.int32),
        pltpu.VMEM((2, KG, SUP, CHUNK), jnp.float32),
        pltpu.VMEM((3, CHUNK, GCOLS), jnp.bfloat16),
        pltpu.VMEM((2, CHUNK, MW), jnp.float32),
        pltpu.VMEM_SHARED((NPAD, MW), jnp.float32),
        pltpu.SemaphoreType.DMA((2,)),
        pltpu.SemaphoreType.DMA((3,)),
        pltpu.SemaphoreType.DMA((2,)),
    ],
)
def _sc_conv(xg_hbm, srcc_hbm, dstc_hbm, w_hbm, agg_out,
             src_v, dst_v, w_v, rows_v, msg_v, agg_sh,
             sem_i, sem_g, sem_s):
    c = lax.axis_index("c")
    s = lax.axis_index("s")

    zf = jnp.zeros((LANES,), jnp.float32)
    of = jnp.ones((LANES,), jnp.float32)

    def zmsg(i, t):
        for j in range(MW // LANES):
            msg_v[0, i, pl.ds(j * LANES, LANES)] = zf
        return t

    lax.fori_loop(0, CHUNK, zmsg, 0)
    row0 = s * ROWS_SUB
    for j in range(ROWS_SUB // CHUNK):
        pltpu.sync_copy(msg_v.at[0],
                        agg_sh.at[pl.ds(row0 + j * CHUNK, CHUNK)])

    def ones_cols(i, t):
        msg_v[0, i, pl.ds(DH, LANES)] = of
        msg_v[1, i, pl.ds(DH, LANES)] = of
        return t

    lax.fori_loop(0, CHUNK, ones_cols, 0)
    plsc.subcore_barrier()

    chunk0 = s * BASE_CPT

    def load_super(sup, buf):
        sl = pl.ds(chunk0 + sup * SUP, SUP)
        pltpu.async_copy(srcc_hbm.at[sl, :], src_v.at[buf], sem_i.at[buf])
        pltpu.async_copy(dstc_hbm.at[sl, :], dst_v.at[buf], sem_i.at[buf])
        pltpu.async_copy(w_hbm.at[:, sl, :], w_v.at[buf], sem_i.at[buf])

    def wait_super(buf):
        pltpu.make_async_copy(srcc_hbm.at[pl.ds(0, SUP), :],
                              src_v.at[buf], sem_i.at[buf]).wait()
        pltpu.make_async_copy(dstc_hbm.at[pl.ds(0, SUP), :],
                              dst_v.at[buf], sem_i.at[buf]).wait()
        pltpu.make_async_copy(w_hbm.at[:, pl.ds(0, SUP), :],
                              w_v.at[buf], sem_i.at[buf]).wait()

    def gather(buf, q, slot):
        pltpu.async_copy(xg_hbm.at[src_v.at[buf, q]], rows_v.at[slot],
                         sem_g.at[slot])

    def wait_gather(slot):
        pltpu.make_async_copy(xg_hbm.at[src_v.at[0, 0]], rows_v.at[slot],
                              sem_g.at[slot]).wait()

    load_super(0, 0)

    def super_body(sup, t):
        cur = lax.rem(sup, 2)
        nxt = lax.rem(sup + 1, 2)
        wait_super(cur)

        coff = lax.broadcast(c * N, (LANES,))

        def rebase(i, t2):
            for g in range(EGROUPS):
                sl = pl.ds(g * LANES, LANES)
                src_v[cur, i, sl] = src_v[cur, i, sl] + coff
            return t2

        lax.fori_loop(0, SUP, rebase, 0)

        @pl.when(sup < NSUP - 1)
        def _():
            load_super(sup + 1, nxt)

        for q in range(SUP):
            if q < 3:
                gather(cur, q, q % 3)

        for q in range(SUP):
            slot = q % 3
            mb = q % 2
            wait_gather(slot)

            if q >= 2:
                pltpu.make_async_copy(
                    msg_v.at[mb], agg_sh.at[dst_v.at[cur, q]],
                    sem_s.at[mb]).wait()
            else:
                @pl.when(sup > 0)
                def _():
                    pltpu.make_async_copy(
                        msg_v.at[mb], agg_sh.at[dst_v.at[cur, q]],
                        sem_s.at[mb]).wait()

            compute_msg(cur, q, slot, mb)

            if q < SUP - 3:
                gather(cur, q + 3, slot)

            pltpu.async_copy(msg_v.at[mb], agg_sh.at[dst_v.at[cur, q]],
                             sem_s.at[mb], add=True)
        return t

    def compute_msg(buf, q, slot, mb):
        def gbody(g, t2):
            e0 = g * LANES
            wvecs = [w_v[buf, k, q, pl.ds(e0, LANES)] for k in range(KG)]
            for i in range(LANES):
                e = e0 + i
                wk = []
                for k in range(KG):
                    ws = lax.broadcast(wvecs[k][i], (LANES,))
                    wk.append(plsc.pack(
                        ws, ws, format=plsc.PackFormat.INTERLEAVED))
                for half in range(2):
                    acc = None
                    for k in range(KG):
                        x32 = rows_v[slot, e,
                                     pl.ds(k * DH + half * 32, 32)]
                        term = x32 * wk[k]
                        acc = term if acc is None else acc + term
                    a, b = plsc.unpack(
                        acc, format=plsc.PackFormat.INTERLEAVED)
                    msg_v[mb, e, pl.ds(2 * half * LANES, LANES)] = a
                    msg_v[mb, e, pl.ds((2 * half + 1) * LANES, LANES)] = b
            return t2

        lax.fori_loop(0, EGROUPS, gbody, 0)

    lax.fori_loop(0, NSUP, super_body, 0)

    for mb in range(2):
        pltpu.make_async_copy(msg_v.at[mb], agg_sh.at[dst_v.at[0, 0]],
                              sem_s.at[mb]).wait()

    @pl.when(s < TAIL)
    def _():
        tc = NS * BASE_CPT + s
        pltpu.sync_copy(srcc_hbm.at[pl.ds(tc, 1), :],
                        src_v.at[0, pl.ds(0, 1), :])
        pltpu.sync_copy(dstc_hbm.at[pl.ds(tc, 1), :],
                        dst_v.at[0, pl.ds(0, 1), :])
        pltpu.sync_copy(w_hbm.at[:, pl.ds(tc, 1), :],
                        w_v.at[0, :, pl.ds(0, 1), :])
        coff = lax.broadcast(c * N, (LANES,))
        for g in range(EGROUPS):
            sl = pl.ds(g * LANES, LANES)
            src_v[0, 0, sl] = src_v[0, 0, sl] + coff
        pltpu.async_copy(xg_hbm.at[src_v.at[0, 0]], rows_v.at[0],
                         sem_g.at[0])
        pltpu.make_async_copy(xg_hbm.at[src_v.at[0, 0]], rows_v.at[0],
                              sem_g.at[0]).wait()
        compute_msg(0, 0, 0, 0)
        pltpu.sync_copy(msg_v.at[0], agg_sh.at[dst_v.at[0, 0]], add=True)

    plsc.subcore_barrier()
    pltpu.sync_copy(agg_sh.at[pl.ds(row0, ROWS_SUB)],
                    agg_out.at[pl.ds(row0, ROWS_SUB), pl.ds(c * MW, MW)])


def kernel(vals, edges, pseudo, g0, mu0, sigma0, root0, bias0, dense0,
           gamma0, beta0, g1, mu1, sigma1, root1, bias1, dense1, gamma1,
           beta1):
    srcc = edges[0].reshape(NCHUNKS, CHUNK)
    dstc = edges[1].reshape(NCHUNKS, CHUNK)
    pseudo_t = pseudo.T.reshape(PDIM, E // 128, 128)

    perm = []
    for cc in range(NC):
        for k in range(KG):
            base = k * D + cc * DH
            for half in range(2):
                for i in range(LANES):
                    perm.append(base + half * 32 + i)
                    perm.append(base + half * 32 + LANES + i)
    perm = jnp.array(perm, dtype=jnp.int32)

    def permute(g):
        return g[:, perm]

    gp0 = permute(g0)
    gp1 = permute(g1)
    xg0, r0, w0, w1 = _tc_pre(vals, gp0, root0, dense0, bias0, pseudo_t,
                              mu0, sigma0, mu1, sigma1)
    agg0 = _sc_conv(xg0.reshape(NC * N, GCOLS), srcc, dstc,
                    w0.reshape(KG, NCHUNKS, CHUNK))
    xg1, r1 = _tc_mid(agg0, r0, gamma0, beta0, gp1, root1, dense1, bias1)
    agg1 = _sc_conv(xg1.reshape(NC * N, GCOLS), srcc, dstc,
                    w1.reshape(KG, NCHUNKS, CHUNK))
    return _tc_post(agg1, r1, gamma1, beta1)
